# baseline (speedup 1.0000x reference)
import functools

import jax
import jax.numpy as jnp
from jax import lax
from jax.experimental import pallas as pl
from jax.experimental.pallas import tpu as pltpu
from jax.experimental.pallas import tpu_sc as plsc

N_NODES = 10000
D = 256
Q = 512
E = 160000

NC = 2
NS = 16
LANES = 16
DH = D // NC
EPS = E // NS
CHUNK = 80
NCHUNKS = EPS // CHUNK
NBUF = 4
RPS = 624
RTAIL = N_NODES - NS * RPS


def _split(x):
    n = x.shape[0]
    return x.reshape(n, NC, DH).transpose(1, 0, 2).reshape(NC * n, DH)


def _merge(x2):
    n = x2.shape[0] // NC
    return x2.reshape(NC, n, DH).transpose(1, 0, 2).reshape(n, NC * DH)


def _pack_idx(src, dst):
    si = src.reshape(-1, CHUNK)
    di = dst.reshape(-1, CHUNK)
    return jnp.stack([si, di], axis=1).reshape(-1)


def _make_spmm():
    mesh = plsc.VectorSubcoreMesh(
        core_axis_name="c", subcore_axis_name="s", num_cores=NC, num_subcores=NS
    )

    @functools.partial(
        pl.kernel,
        out_type=jax.ShapeDtypeStruct((NC * N_NODES, DH), jnp.float32),
        mesh=mesh,
        scratch_types=[
            [pltpu.VMEM((2 * CHUNK,), jnp.int32) for _ in range(NBUF)],
            [pltpu.VMEM((CHUNK,), jnp.float32) for _ in range(NBUF)],
            [pltpu.VMEM((CHUNK,), jnp.int32) for _ in range(NBUF)],
            [pltpu.VMEM((CHUNK,), jnp.int32) for _ in range(NBUF)],
            [pltpu.VMEM((CHUNK, DH), jnp.float32) for _ in range(NBUF)],
            pltpu.VMEM_SHARED((N_NODES, DH), jnp.float32),
            [pltpu.SemaphoreType.DMA for _ in range(NBUF)],
            [pltpu.SemaphoreType.DMA for _ in range(NBUF)],
            [pltpu.SemaphoreType.DMA for _ in range(NBUF)],
        ],
    )
    def spmm(packed_hbm, val_hbm, x2_hbm, init2_hbm, out_hbm,
             ebuf, valb, srcb, dstb, rows, acc, esem, gsem, ssem):
        c = lax.axis_index("c")
        s = lax.axis_index("s")
        off = c * N_NODES
        cbase = s * NCHUNKS

        pltpu.sync_copy(init2_hbm.at[pl.ds(off + s * RPS, RPS)],
                        acc.at[pl.ds(s * RPS, RPS)])

        @pl.when(s == NS - 1)
        def _():
            pltpu.sync_copy(init2_hbm.at[pl.ds(off + NS * RPS, RTAIL)],
                            acc.at[pl.ds(NS * RPS, RTAIL)])

        def issue_edges(t, b):
            eb = (cbase + t) * (2 * CHUNK)
            pltpu.async_copy(packed_hbm.at[pl.ds(eb, 2 * CHUNK)], ebuf[b],
                             esem[b])
            pltpu.async_copy(val_hbm.at[pl.ds((cbase + t) * CHUNK, CHUNK)],
                             valb[b], esem[b])

        def drain_edges(b):
            pltpu.make_async_copy(packed_hbm.at[pl.ds(0, 2 * CHUNK)], ebuf[b],
                                  esem[b]).wait()
            pltpu.make_async_copy(val_hbm.at[pl.ds(0, CHUNK)], valb[b],
                                  esem[b]).wait()

        def issue_gather(b):
            for i in range(CHUNK // LANES):
                sl = pl.ds(i * LANES, LANES)
                srcb[b][sl] = ebuf[b][pl.ds(i * LANES, LANES)] + off
                dstb[b][sl] = ebuf[b][pl.ds(CHUNK + i * LANES, LANES)]
            pltpu.async_copy(x2_hbm.at[srcb[b]], rows[b], gsem[b])

        def drain_rows(b, sem):
            pltpu.make_async_copy(init2_hbm.at[pl.ds(0, CHUNK)],
                                  rows[b], sem[b]).wait()

        def scale_and_scatter(b):
            def scale(g, cc):
                vv16 = valb[b][pl.ds(g * LANES, LANES)]
                for e16 in range(LANES):
                    splat = jnp.full((LANES,), vv16[e16])
                    row = g * LANES + e16
                    for jj in range(DH // LANES):
                        sl = pl.ds(jj * LANES, LANES)
                        rows[b][row, sl] = rows[b][row, sl] * splat
                return cc

            lax.fori_loop(0, CHUNK // LANES, scale, 0)
            pltpu.async_copy(rows[b], acc.at[dstb[b]], ssem[b], add=True)

        plsc.subcore_barrier()

        for t0 in range(3):
            issue_edges(t0, t0)
        for t0 in range(2):
            drain_edges(t0)
            issue_gather(t0)

        def step(t, b):
            @pl.when(t >= 1)
            def _():
                drain_rows((b + NBUF - 1) % NBUF, ssem)

            @pl.when(t + 2 < NCHUNKS)
            def _():
                bg = (b + 2) % NBUF
                drain_edges(bg)
                issue_gather(bg)

            drain_rows(b, gsem)
            scale_and_scatter(b)

            @pl.when(t + 3 < NCHUNKS)
            def _():
                issue_edges(t + 3, (b + 3) % NBUF)

        def outer(j, carry):
            for b in range(NBUF):
                step(j * NBUF + b, b)
            return carry

        lax.fori_loop(0, (NCHUNKS - 1) // NBUF, outer, 0)
        step(jnp.int32(NCHUNKS - 1), (NCHUNKS - 1) % NBUF)

        drain_rows((NCHUNKS - 1) % NBUF, ssem)
        plsc.subcore_barrier()

        pltpu.sync_copy(acc.at[pl.ds(s * RPS, RPS)],
                        out_hbm.at[pl.ds(off + s * RPS, RPS)])

        @pl.when(s == NS - 1)
        def _():
            pltpu.sync_copy(acc.at[pl.ds(NS * RPS, RTAIL)],
                            out_hbm.at[pl.ds(NS * RPS + off, RTAIL)])

    return spmm


_spmm = _make_spmm()


def _mm1_body(a_ref, b_ref, o_ref):
    o_ref[0] = jnp.dot(a_ref[...], b_ref[0],
                       preferred_element_type=jnp.float32)


def _mm1(a, b2):
    b3 = b2.reshape(NC, N_NODES, DH)
    return pl.pallas_call(
        _mm1_body,
        grid=(NC,),
        in_specs=[
            pl.BlockSpec((Q, N_NODES), lambda c: (0, 0)),
            pl.BlockSpec((1, N_NODES, DH), lambda c: (c, 0, 0)),
        ],
        out_specs=pl.BlockSpec((1, Q, DH), lambda c: (c, 0, 0)),
        out_shape=jax.ShapeDtypeStruct((NC, Q, DH), jnp.float32),
        compiler_params=pltpu.CompilerParams(
            dimension_semantics=("arbitrary",),
        ),
    )(a, b3)


_MB = 1000


def _mm2_body(a_ref, b_ref, add_ref, o_ref):
    o_ref[...] = add_ref[...] + jnp.dot(
        a_ref[...], b_ref[0], preferred_element_type=jnp.float32
    )


def _mm2(a, t, add):
    return pl.pallas_call(
        _mm2_body,
        grid=(N_NODES // _MB, NC),
        in_specs=[
            pl.BlockSpec((_MB, Q), lambda m, c: (m, 0)),
            pl.BlockSpec((1, Q, DH), lambda m, c: (c, 0, 0)),
            pl.BlockSpec((_MB, DH), lambda m, c: (m, c)),
        ],
        out_specs=pl.BlockSpec((_MB, DH), lambda m, c: (m, c)),
        out_shape=jax.ShapeDtypeStruct((N_NODES, D), jnp.float32),
        compiler_params=pltpu.CompilerParams(
            dimension_semantics=("parallel", "parallel"),
        ),
    )(a, t, add)


@jax.jit
def kernel(adj_rows, adj_cols, adj_vals, E_u_0, E_i_0, u_mul_s, v_mul_s, ut, vt):
    rows = adj_rows.astype(jnp.int32)
    cols = adj_cols.astype(jnp.int32)

    eu0_2 = _split(E_u_0)
    ei0_2 = _split(E_i_0)
    pk_u = _pack_idx(cols, rows)
    pk_i = _pack_idx(rows, cols)

    hu_2 = _spmm(pk_u, adj_vals, ei0_2, eu0_2)
    hi_2 = _spmm(pk_i, adj_vals, eu0_2, ei0_2)

    eu_sum_2 = _spmm(pk_u, adj_vals, hi_2, eu0_2)
    ei_sum_2 = _spmm(pk_i, adj_vals, hu_2, ei0_2)

    t_u = _mm1(vt, hi_2)
    g_u_sum = _mm2(u_mul_s, t_u, E_u_0)
    t_i = _mm1(ut, hu_2)
    g_i_sum = _mm2(v_mul_s, t_i, E_i_0)

    return (g_u_sum, g_i_sum, _merge(eu_sum_2), _merge(ei_sum_2))


# speedup vs baseline: 1.1906x; 1.1906x over previous
import functools

import jax
import jax.numpy as jnp
from jax import lax
from jax.experimental import pallas as pl
from jax.experimental.pallas import tpu as pltpu
from jax.experimental.pallas import tpu_sc as plsc

N_NODES = 10000
D = 256
Q = 512
E = 160000

NC = 2
NS = 16
LANES = 16
DH = D // NC
EPS = E // NS
CHUNK = 80
NCHUNKS = EPS // CHUNK
NBUF = 4
RPS = 624
RTAIL = N_NODES - NS * RPS


def _split(x):
    n = x.shape[0]
    return x.reshape(n, NC, DH).transpose(1, 0, 2).reshape(NC * n, DH)


def _merge(x2):
    n = x2.shape[0] // NC
    return x2.reshape(NC, n, DH).transpose(1, 0, 2).reshape(n, NC * DH)


def _pack_idx(src, dst):
    si = src.reshape(-1, CHUNK)
    di = dst.reshape(-1, CHUNK)
    return jnp.stack([si, di], axis=1).reshape(-1)


def _make_spmm():
    mesh = plsc.VectorSubcoreMesh(
        core_axis_name="c", subcore_axis_name="s", num_cores=NC, num_subcores=NS
    )

    @functools.partial(
        pl.kernel,
        out_type=jax.ShapeDtypeStruct((NC * N_NODES, DH), jnp.float32),
        mesh=mesh,
        scratch_types=[
            [pltpu.VMEM((2 * CHUNK,), jnp.int32) for _ in range(NBUF)],
            [pltpu.VMEM((CHUNK,), jnp.float32) for _ in range(NBUF)],
            [pltpu.VMEM((CHUNK,), jnp.int32) for _ in range(NBUF)],
            [pltpu.VMEM((CHUNK,), jnp.int32) for _ in range(NBUF)],
            [pltpu.VMEM((CHUNK, DH), jnp.float32) for _ in range(NBUF)],
            pltpu.VMEM_SHARED((N_NODES, DH), jnp.float32),
            [pltpu.SemaphoreType.DMA for _ in range(NBUF)],
            [pltpu.SemaphoreType.DMA for _ in range(NBUF)],
            [pltpu.SemaphoreType.DMA for _ in range(NBUF)],
        ],
    )
    def spmm(packed_hbm, val_hbm, x2_hbm, init2_hbm, out_hbm,
             ebuf, valb, srcb, dstb, rows, acc, esem, gsem, ssem):
        c = lax.axis_index("c")
        s = lax.axis_index("s")
        off = c * N_NODES
        cbase = s * NCHUNKS

        pltpu.sync_copy(init2_hbm.at[pl.ds(off + s * RPS, RPS)],
                        acc.at[pl.ds(s * RPS, RPS)])

        @pl.when(s == NS - 1)
        def _():
            pltpu.sync_copy(init2_hbm.at[pl.ds(off + NS * RPS, RTAIL)],
                            acc.at[pl.ds(NS * RPS, RTAIL)])

        def issue_edges(t, b):
            eb = (cbase + t) * (2 * CHUNK)
            pltpu.async_copy(packed_hbm.at[pl.ds(eb, 2 * CHUNK)], ebuf[b],
                             esem[b])
            pltpu.async_copy(val_hbm.at[pl.ds((cbase + t) * CHUNK, CHUNK)],
                             valb[b], esem[b])

        def drain_edges(b):
            pltpu.make_async_copy(packed_hbm.at[pl.ds(0, 2 * CHUNK)], ebuf[b],
                                  esem[b]).wait()
            pltpu.make_async_copy(val_hbm.at[pl.ds(0, CHUNK)], valb[b],
                                  esem[b]).wait()

        def issue_gather(b):
            for i in range(CHUNK // LANES):
                sl = pl.ds(i * LANES, LANES)
                srcb[b][sl] = ebuf[b][pl.ds(i * LANES, LANES)] + off
                dstb[b][sl] = ebuf[b][pl.ds(CHUNK + i * LANES, LANES)]
            pltpu.async_copy(x2_hbm.at[srcb[b]], rows[b], gsem[b])

        def drain_rows(b, sem):
            pltpu.make_async_copy(init2_hbm.at[pl.ds(0, CHUNK)],
                                  rows[b], sem[b]).wait()

        def scale_and_scatter(b):
            def scale(g, cc):
                vv16 = valb[b][pl.ds(g * LANES, LANES)]
                for e16 in range(LANES):
                    splat = jnp.full((LANES,), vv16[e16])
                    row = g * LANES + e16
                    for jj in range(DH // LANES):
                        sl = pl.ds(jj * LANES, LANES)
                        rows[b][row, sl] = rows[b][row, sl] * splat
                return cc

            lax.fori_loop(0, CHUNK // LANES, scale, 0)
            pltpu.async_copy(rows[b], acc.at[dstb[b]], ssem[b], add=True)

        plsc.subcore_barrier()

        for t0 in range(3):
            issue_edges(t0, t0)
        for t0 in range(2):
            drain_edges(t0)
            issue_gather(t0)

        def step(t, b):
            @pl.when(t + 3 < NCHUNKS)
            def _():
                issue_edges(t + 3, (b + 3) % NBUF)

            @pl.when(t >= 2)
            def _():
                drain_rows((b + 2) % NBUF, ssem)

            @pl.when(t + 2 < NCHUNKS)
            def _():
                bg = (b + 2) % NBUF
                drain_edges(bg)
                issue_gather(bg)

            drain_rows(b, gsem)
            scale_and_scatter(b)

        def outer(j, carry):
            for b in range(NBUF):
                step(j * NBUF + b, b)
            return carry

        lax.fori_loop(0, (NCHUNKS - 1) // NBUF, outer, 0)
        step(jnp.int32(NCHUNKS - 1), (NCHUNKS - 1) % NBUF)

        drain_rows((NCHUNKS - 2) % NBUF, ssem)
        drain_rows((NCHUNKS - 1) % NBUF, ssem)
        plsc.subcore_barrier()

        pltpu.sync_copy(acc.at[pl.ds(s * RPS, RPS)],
                        out_hbm.at[pl.ds(off + s * RPS, RPS)])

        @pl.when(s == NS - 1)
        def _():
            pltpu.sync_copy(acc.at[pl.ds(NS * RPS, RTAIL)],
                            out_hbm.at[pl.ds(NS * RPS + off, RTAIL)])

    return spmm


_spmm = _make_spmm()


def _mm1_body(a_ref, b_ref, o_ref):
    o_ref[0] = jnp.dot(a_ref[...], b_ref[0],
                       preferred_element_type=jnp.float32)


def _mm1(a, b2):
    b3 = b2.reshape(NC, N_NODES, DH)
    return pl.pallas_call(
        _mm1_body,
        grid=(NC,),
        in_specs=[
            pl.BlockSpec((Q, N_NODES), lambda c: (0, 0)),
            pl.BlockSpec((1, N_NODES, DH), lambda c: (c, 0, 0)),
        ],
        out_specs=pl.BlockSpec((1, Q, DH), lambda c: (c, 0, 0)),
        out_shape=jax.ShapeDtypeStruct((NC, Q, DH), jnp.float32),
        compiler_params=pltpu.CompilerParams(
            dimension_semantics=("arbitrary",),
        ),
    )(a, b3)


_MB = 1000


def _mm2_body(a_ref, b_ref, add_ref, o_ref):
    o_ref[...] = add_ref[...] + jnp.dot(
        a_ref[...], b_ref[0], preferred_element_type=jnp.float32
    )


def _mm2(a, t, add):
    return pl.pallas_call(
        _mm2_body,
        grid=(N_NODES // _MB, NC),
        in_specs=[
            pl.BlockSpec((_MB, Q), lambda m, c: (m, 0)),
            pl.BlockSpec((1, Q, DH), lambda m, c: (c, 0, 0)),
            pl.BlockSpec((_MB, DH), lambda m, c: (m, c)),
        ],
        out_specs=pl.BlockSpec((_MB, DH), lambda m, c: (m, c)),
        out_shape=jax.ShapeDtypeStruct((N_NODES, D), jnp.float32),
        compiler_params=pltpu.CompilerParams(
            dimension_semantics=("parallel", "parallel"),
        ),
    )(a, t, add)


@jax.jit
def kernel(adj_rows, adj_cols, adj_vals, E_u_0, E_i_0, u_mul_s, v_mul_s, ut, vt):
    rows = adj_rows.astype(jnp.int32)
    cols = adj_cols.astype(jnp.int32)

    eu0_2 = _split(E_u_0)
    ei0_2 = _split(E_i_0)
    pk_u = _pack_idx(cols, rows)
    pk_i = _pack_idx(rows, cols)

    hu_2 = _spmm(pk_u, adj_vals, ei0_2, eu0_2)
    hi_2 = _spmm(pk_i, adj_vals, eu0_2, ei0_2)

    eu_sum_2 = _spmm(pk_u, adj_vals, hi_2, eu0_2)
    ei_sum_2 = _spmm(pk_i, adj_vals, hu_2, ei0_2)

    t_u = _mm1(vt, hi_2)
    g_u_sum = _mm2(u_mul_s, t_u, E_u_0)
    t_i = _mm1(ut, hu_2)
    g_i_sum = _mm2(v_mul_s, t_i, E_i_0)

    return (g_u_sum, g_i_sum, _merge(eu_sum_2), _merge(ei_sum_2))
